# baseline (speedup 1.0000x reference)
import jax
import jax.numpy as jnp
from jax.experimental import pallas as pl
from jax.experimental.pallas import tpu as pltpu

_T, _N, _D = 4, 50000, 128
_ROWS = _T * _N
_BLK = 10000
_GRID = _ROWS // _BLK


def _spike_kernel(obs_ref, nd_ref, pos_ref, neg_ref, out_ref):
    obs = obs_ref[0]
    col = jnp.reshape(obs, (_BLK, 1))
    pos_m = (col == 1).astype(jnp.float32)
    neg_m = (col == -1).astype(jnp.float32)
    spike = pos_m * pos_ref[...] + neg_m * neg_ref[...]
    out_ref[...] = nd_ref[...] + spike


def kernel(node_data, observations, pos_test_spike, neg_test_spike):
    nd = node_data.reshape(_ROWS, _D)
    obs = observations.reshape(_GRID, 1, _BLK).astype(jnp.int32)
    pos = pos_test_spike.reshape(1, _D)
    neg = neg_test_spike.reshape(1, _D)

    out = pl.pallas_call(
        _spike_kernel,
        grid=(_GRID,),
        in_specs=[
            pl.BlockSpec((1, 1, _BLK), lambda i: (0, i, 0)),
            pl.BlockSpec((_BLK, _D), lambda i: (i, 0)),
            pl.BlockSpec((1, _D), lambda i: (0, 0)),
            pl.BlockSpec((1, _D), lambda i: (0, 0)),
        ],
        out_specs=pl.BlockSpec((_BLK, _D), lambda i: (i, 0)),
        out_shape=jax.ShapeDtypeStruct((_ROWS, _D), jnp.float32),
        compiler_params=pltpu.CompilerParams(
            dimension_semantics=("parallel",)),
    )(obs, nd, pos, neg)
    return out.reshape(_T, _N, _D)


# speedup vs baseline: 1.0164x; 1.0164x over previous
import jax
import jax.numpy as jnp
from jax.experimental import pallas as pl
from jax.experimental.pallas import tpu as pltpu

_T, _N, _D = 4, 50000, 128
_ROWS = _T * _N
_BLK = 25000
_GRID = _ROWS // _BLK


def _spike_kernel(obs_ref, nd_ref, pos_ref, neg_ref, out_ref):
    obs = obs_ref[0]
    col = jnp.reshape(obs, (_BLK, 1))
    pos_m = (col == 1).astype(jnp.float32)
    neg_m = (col == -1).astype(jnp.float32)
    spike = pos_m * pos_ref[...] + neg_m * neg_ref[...]
    out_ref[...] = nd_ref[...] + spike


def kernel(node_data, observations, pos_test_spike, neg_test_spike):
    nd = node_data.reshape(_ROWS, _D)
    obs = observations.reshape(_GRID, 1, _BLK).astype(jnp.int32)
    pos = pos_test_spike.reshape(1, _D)
    neg = neg_test_spike.reshape(1, _D)

    out = pl.pallas_call(
        _spike_kernel,
        grid=(_GRID,),
        in_specs=[
            pl.BlockSpec((1, 1, _BLK), lambda i: (0, i, 0)),
            pl.BlockSpec((_BLK, _D), lambda i: (i, 0)),
            pl.BlockSpec((1, _D), lambda i: (0, 0)),
            pl.BlockSpec((1, _D), lambda i: (0, 0)),
        ],
        out_specs=pl.BlockSpec((_BLK, _D), lambda i: (i, 0)),
        out_shape=jax.ShapeDtypeStruct((_ROWS, _D), jnp.float32),
        compiler_params=pltpu.CompilerParams(
            dimension_semantics=("parallel",)),
    )(obs, nd, pos, neg)
    return out.reshape(_T, _N, _D)
